# baseline (speedup 1.0000x reference)
import functools

import jax
import jax.numpy as jnp
from jax import lax
from jax.experimental import pallas as pl
from jax.experimental.pallas import tpu as pltpu
from jax.experimental.pallas import tpu_sc as plsc


def _embed_bag_sc(C, K, A, D):
  info = plsc.get_sparse_core_info()
  nw = info.num_cores * info.num_subcores
  per_w = C // nw
  G = 5
  n_groups = per_w // G
  STAGE = 25
  n_stages = n_groups // STAGE
  rows_per_stage = STAGE * G

  offs = list(range(0, D - 15, 16))
  if D % 16:
    offs.append(D - 16)

  mesh = plsc.VectorSubcoreMesh(core_axis_name="c", subcore_axis_name="s")

  @functools.partial(
      pl.kernel,
      mesh=mesh,
      compiler_params=pltpu.CompilerParams(use_tc_tiling_on_sc=False),
      out_type=jax.ShapeDtypeStruct((C * D,), jnp.float32),
      scratch_types=[
          pltpu.VMEM((per_w * K,), jnp.int32),
          pltpu.VMEM((G * K, D), jnp.float32),
          pltpu.VMEM((rows_per_stage * D,), jnp.float32),
          pltpu.SemaphoreType.DMA,
      ],
  )
  def bag(idx_hbm, table_hbm, out_hbm, idx_v, buf_v, out_v, sem):
    wid = lax.axis_index("s") * info.num_cores + lax.axis_index("c")
    c0 = wid * per_w
    pltpu.sync_copy(idx_hbm.at[pl.ds(c0 * K, per_w * K)], idx_v)

    def do_stage(s, _):
      def do_group(gs, _):
        goff = s * STAGE + gs
        pltpu.async_copy(
            table_hbm.at[idx_v.at[pl.ds(goff * (G * K), G * K)]], buf_v, sem
        ).wait()
        for i in range(G):
          for o in offs:
            acc = buf_v[i * K, pl.ds(o, 16)]
            for kk in range(1, K):
              acc = acc + buf_v[i * K + kk, pl.ds(o, 16)]
            out_v[pl.ds((gs * G + i) * D + o, 16)] = acc
        return 0

      lax.fori_loop(0, STAGE, do_group, 0)
      pltpu.sync_copy(
          out_v,
          out_hbm.at[pl.ds((c0 + s * rows_per_stage) * D, rows_per_stage * D)],
      )
      return 0

    lax.fori_loop(0, n_stages, do_stage, 0)

  return bag


def _matmul_softmax_tc(x, h, bb):
  B, D = x.shape
  C = h.shape[0]

  def body(x_ref, h_ref, o_ref):
    logits = lax.dot_general(
        x_ref[...], h_ref[...], (((1,), (1,)), ((), ())),
        preferred_element_type=jnp.float32)
    m = jnp.max(logits, axis=1, keepdims=True)
    e = jnp.exp(logits - m)
    o_ref[...] = e * (1.0 / jnp.sum(e, axis=1, keepdims=True))

  return pl.pallas_call(
      body,
      grid=(B // bb,),
      in_specs=[
          pl.BlockSpec((bb, D), lambda i: (i, 0)),
          pl.BlockSpec((C, D), lambda i: (0, 0)),
      ],
      out_specs=pl.BlockSpec((bb, C), lambda i: (i, 0)),
      out_shape=jax.ShapeDtypeStruct((B, C), jnp.float32),
  )(x, h)


def kernel(x, ancestor_idx, table):
  C, K = ancestor_idx.shape
  A, D = table.shape
  idx = ancestor_idx.astype(jnp.int32).reshape(-1)
  local_h = _embed_bag_sc(C, K, A, D)(idx, table).reshape(C, D)
  return _matmul_softmax_tc(x, local_h, 128)


# speedup vs baseline: 1.0274x; 1.0274x over previous
import functools

import jax
import jax.numpy as jnp
from jax import lax
from jax.experimental import pallas as pl
from jax.experimental.pallas import tpu as pltpu
from jax.experimental.pallas import tpu_sc as plsc


def _embed_bag_sc(C, K, A, Dp):
  info = plsc.get_sparse_core_info()
  nw = info.num_cores * info.num_subcores
  per_w = C // nw
  G = 5
  GK = G * K
  n_groups = per_w // G
  STAGE = 25
  rows_stage = STAGE * G

  mesh = plsc.VectorSubcoreMesh(core_axis_name="c", subcore_axis_name="s")

  @functools.partial(
      pl.kernel,
      mesh=mesh,
      out_type=jax.ShapeDtypeStruct((C * Dp,), jnp.float32),
      scratch_types=[
          pltpu.VMEM((per_w * K,), jnp.int32),
          pltpu.VMEM((GK, Dp), jnp.float32),
          pltpu.VMEM((GK, Dp), jnp.float32),
          pltpu.VMEM((rows_stage * Dp,), jnp.float32),
          pltpu.SemaphoreType.DMA,
          pltpu.SemaphoreType.DMA,
      ],
  )
  def bag(idx_hbm, table_hbm, out_hbm, idx_v, buf0, buf1, out_v, sem0, sem1):
    wid = lax.axis_index("s") * info.num_cores + lax.axis_index("c")
    c0 = wid * per_w
    pltpu.sync_copy(idx_hbm.at[pl.ds(c0 * K, per_w * K)], idx_v)

    def fire(g, buf, sem):
      pltpu.async_copy(table_hbm.at[idx_v.at[pl.ds(g * GK, GK)]], buf, sem)

    def wait(buf, sem):
      pltpu.make_async_copy(
          table_hbm.at[idx_v.at[pl.ds(0, GK)]], buf, sem).wait()

    def reduce_group(g, buf):
      rb = (g % STAGE) * G
      for i in range(G):
        for o in range(0, Dp, 16):
          acc = buf[i * K, pl.ds(o, 16)]
          for kk in range(1, K):
            acc = acc + buf[i * K + kk, pl.ds(o, 16)]
          out_v[pl.ds((rb + i) * Dp + o, 16)] = acc

    def maybe_writeback(g):
      @pl.when((g + 1) % STAGE == 0)
      def _():
        st = g // STAGE
        pltpu.sync_copy(
            out_v,
            out_hbm.at[pl.ds((c0 + st * rows_stage) * Dp, rows_stage * Dp)],
        )

    fire(0, buf0, sem0)
    fire(1, buf1, sem1)

    def pair(i, _):
      g0 = 2 * i
      wait(buf0, sem0)
      reduce_group(g0, buf0)
      maybe_writeback(g0)
      fire(g0 + 2, buf0, sem0)

      g1 = 2 * i + 1
      wait(buf1, sem1)
      reduce_group(g1, buf1)
      maybe_writeback(g1)

      @pl.when(g1 + 2 < n_groups)
      def _():
        fire(g1 + 2, buf1, sem1)
      return 0

    lax.fori_loop(0, (n_groups - 1) // 2, pair, 0)

    g_last = n_groups - 1
    wait(buf0, sem0)
    reduce_group(g_last, buf0)
    maybe_writeback(g_last)

  return bag


def _matmul_softmax_tc(x, h, bb):
  B, Dp = x.shape
  C = h.shape[0]

  def body(x_ref, h_ref, o_ref):
    logits = lax.dot_general(
        x_ref[...], h_ref[...], (((1,), (1,)), ((), ())),
        preferred_element_type=jnp.float32)
    m = jnp.max(logits, axis=1, keepdims=True)
    e = jnp.exp(logits - m)
    o_ref[...] = e * (1.0 / jnp.sum(e, axis=1, keepdims=True))

  return pl.pallas_call(
      body,
      grid=(B // bb,),
      in_specs=[
          pl.BlockSpec((bb, Dp), lambda i: (i, 0)),
          pl.BlockSpec((C, Dp), lambda i: (0, 0)),
      ],
      out_specs=pl.BlockSpec((bb, C), lambda i: (i, 0)),
      out_shape=jax.ShapeDtypeStruct((B, C), jnp.float32),
  )(x, h)


def kernel(x, ancestor_idx, table):
  C, K = ancestor_idx.shape
  A, D = table.shape
  Dp = 256
  idx = ancestor_idx.astype(jnp.int32).reshape(-1)
  table_p = jnp.pad(table, ((0, 0), (0, Dp - D)))
  x_p = jnp.pad(x, ((0, 0), (0, Dp - D)))
  local_h = _embed_bag_sc(C, K, A, Dp)(idx, table_p).reshape(C, Dp)
  return _matmul_softmax_tc(x_p, local_h, 128)


# speedup vs baseline: 1.6971x; 1.6518x over previous
import functools

import jax
import jax.numpy as jnp
from jax import lax
from jax.experimental import pallas as pl
from jax.experimental.pallas import tpu as pltpu
from jax.experimental.pallas import tpu_sc as plsc

_DP = 256
_G = 8


def _pad_table_tc(table, Dp):
  A, D = table.shape
  bs = 4000

  def body(t_ref, o_ref):
    o_ref[...] = jnp.concatenate(
        [t_ref[...], jnp.zeros((bs, Dp - D), jnp.float32)], axis=1)

  return pl.pallas_call(
      body,
      grid=(A // bs,),
      in_specs=[pl.BlockSpec((bs, D), lambda i: (i, 0))],
      out_specs=pl.BlockSpec((bs, Dp), lambda i: (i, 0)),
      out_shape=jax.ShapeDtypeStruct((A, Dp), jnp.float32),
  )(table)


def _embed_bag_sc(C, K, A, Dp):
  info = plsc.get_sparse_core_info()
  nw = info.num_cores * info.num_subcores
  G = _G
  GK = G * K
  n_groups = C // G
  max_w = (n_groups + nw - 1) // nw

  mesh = plsc.VectorSubcoreMesh(core_axis_name="c", subcore_axis_name="s")

  @functools.partial(
      pl.kernel,
      mesh=mesh,
      out_type=jax.ShapeDtypeStruct((C * Dp,), jnp.float32),
      scratch_types=[
          pltpu.VMEM((max_w * GK,), jnp.int32),
          pltpu.VMEM((GK, Dp), jnp.float32),
          pltpu.VMEM((GK, Dp), jnp.float32),
          pltpu.VMEM((G * Dp,), jnp.float32),
          pltpu.SemaphoreType.DMA,
          pltpu.SemaphoreType.DMA,
      ],
  )
  def bag(idx_hbm, table_hbm, out_hbm, idx_v, buf0, buf1, acc_v, sem0, sem1):
    wid = lax.axis_index("s") * info.num_cores + lax.axis_index("c")
    g0 = wid * n_groups // nw
    g1 = (wid + 1) * n_groups // nw
    n_w = g1 - g0
    pltpu.sync_copy(idx_hbm.at[pl.ds(g0 * GK, max_w * GK)], idx_v)

    def fire(s, buf, sem):
      @pl.when(s < n_w)
      def _():
        pltpu.async_copy(table_hbm.at[idx_v.at[pl.ds(s * GK, GK)]], buf, sem)

    def wait(buf, sem):
      pltpu.make_async_copy(
          table_hbm.at[idx_v.at[pl.ds(0, GK)]], buf, sem).wait()

    def process(s, buf):
      for i in range(G):
        for o in range(0, Dp, 16):
          acc = buf[i * K, pl.ds(o, 16)]
          for kk in range(1, K):
            acc = acc + buf[i * K + kk, pl.ds(o, 16)]
          acc_v[pl.ds(i * Dp + o, 16)] = acc
      pltpu.sync_copy(acc_v, out_hbm.at[pl.ds((g0 + s) * (G * Dp), G * Dp)])

    fire(0, buf0, sem0)
    fire(1, buf1, sem1)

    def pair(p, _):
      s0 = 2 * p
      wait(buf0, sem0)
      process(s0, buf0)
      fire(s0 + 2, buf0, sem0)

      s1 = 2 * p + 1
      wait(buf1, sem1)
      process(s1, buf1)
      fire(s1 + 2, buf1, sem1)
      return 0

    lax.fori_loop(0, n_w // 2, pair, 0)

    @pl.when(n_w % 2 == 1)
    def _():
      wait(buf0, sem0)
      process(n_w - 1, buf0)

  return bag


def _matmul_softmax_tc(x, h, bb):
  B, Dp = x.shape
  C = h.shape[0]

  def body(x_ref, h_ref, o_ref):
    logits = lax.dot_general(
        x_ref[...], h_ref[...], (((1,), (1,)), ((), ())),
        preferred_element_type=jnp.float32)
    m = jnp.max(logits, axis=1, keepdims=True)
    e = jnp.exp(logits - m)
    o_ref[...] = e * (1.0 / jnp.sum(e, axis=1, keepdims=True))

  return pl.pallas_call(
      body,
      grid=(B // bb,),
      in_specs=[
          pl.BlockSpec((bb, Dp), lambda i: (i, 0)),
          pl.BlockSpec((C, Dp), lambda i: (0, 0)),
      ],
      out_specs=pl.BlockSpec((bb, C), lambda i: (i, 0)),
      out_shape=jax.ShapeDtypeStruct((B, C), jnp.float32),
  )(x, h)


def kernel(x, ancestor_idx, table):
  C, K = ancestor_idx.shape
  A, D = table.shape
  idx = ancestor_idx.astype(jnp.int32).reshape(-1)
  table_p = _pad_table_tc(table, _DP)
  x_p = jnp.pad(x, ((0, 0), (0, _DP - D)))
  local_h = _embed_bag_sc(C, K, A, _DP)(idx, table_p).reshape(C, _DP)
  return _matmul_softmax_tc(x_p, local_h, 128)


# speedup vs baseline: 2.1441x; 1.2634x over previous
import functools

import jax
import jax.numpy as jnp
from jax import lax
from jax.experimental import pallas as pl
from jax.experimental.pallas import tpu as pltpu
from jax.experimental.pallas import tpu_sc as plsc

_DP = 256
_G = 16


def _pad_table_tc(table, Dp):
  A, D = table.shape
  bs = 4000

  def body(t_ref, o_ref):
    o_ref[...] = jnp.concatenate(
        [t_ref[...], jnp.zeros((bs, Dp - D), jnp.float32)], axis=1)

  return pl.pallas_call(
      body,
      grid=(A // bs,),
      in_specs=[pl.BlockSpec((bs, D), lambda i: (i, 0))],
      out_specs=pl.BlockSpec((bs, Dp), lambda i: (i, 0)),
      out_shape=jax.ShapeDtypeStruct((A, Dp), jnp.float32),
  )(table)


def _embed_bag_sc(C, K, A, Dp):
  info = plsc.get_sparse_core_info()
  nw = info.num_cores * info.num_subcores
  G = _G
  GK = G * K
  n_groups = C // G
  max_w = (n_groups + nw - 1) // nw

  mesh = plsc.VectorSubcoreMesh(core_axis_name="c", subcore_axis_name="s")

  @functools.partial(
      pl.kernel,
      mesh=mesh,
      out_type=jax.ShapeDtypeStruct((C, Dp), jnp.float32),
      scratch_types=[
          pltpu.VMEM((max_w * GK,), jnp.int32),
          pltpu.VMEM((GK, Dp), jnp.float32),
          pltpu.VMEM((GK, Dp), jnp.float32),
          pltpu.VMEM((G, Dp), jnp.float32),
          pltpu.SemaphoreType.DMA,
          pltpu.SemaphoreType.DMA,
      ],
  )
  def bag(idx_hbm, table_hbm, out_hbm, idx_v, buf0, buf1, acc_v, sem0, sem1):
    wid = lax.axis_index("s") * info.num_cores + lax.axis_index("c")
    g0 = wid * n_groups // nw
    g1 = (wid + 1) * n_groups // nw
    n_w = g1 - g0
    pltpu.sync_copy(idx_hbm.at[pl.ds(g0 * GK, max_w * GK)], idx_v)

    def fire(s, buf, sem):
      @pl.when(s < n_w)
      def _():
        pltpu.async_copy(table_hbm.at[idx_v.at[pl.ds(s * GK, GK)]], buf, sem)

    def wait(buf, sem):
      pltpu.make_async_copy(
          table_hbm.at[idx_v.at[pl.ds(0, GK)]], buf, sem).wait()

    def process(s, buf):
      def per_concept(i, _):
        for o in range(0, Dp, 16):
          acc = buf[i * K, pl.ds(o, 16)]
          for kk in range(1, K):
            acc = acc + buf[i * K + kk, pl.ds(o, 16)]
          acc_v[i, pl.ds(o, 16)] = acc
        return 0

      lax.fori_loop(0, G, per_concept, 0)
      pltpu.sync_copy(acc_v, out_hbm.at[pl.ds((g0 + s) * G, G)])

    fire(0, buf0, sem0)
    fire(1, buf1, sem1)

    def pair(p, _):
      s0 = 2 * p
      wait(buf0, sem0)
      process(s0, buf0)
      fire(s0 + 2, buf0, sem0)

      s1 = 2 * p + 1
      wait(buf1, sem1)
      process(s1, buf1)
      fire(s1 + 2, buf1, sem1)
      return 0

    lax.fori_loop(0, n_w // 2, pair, 0)

    @pl.when(n_w % 2 == 1)
    def _():
      wait(buf0, sem0)
      process(n_w - 1, buf0)

  return bag


def _matmul_softmax_tc(x, h, bb):
  B, Dp = x.shape
  C = h.shape[0]

  def body(x_ref, h_ref, o_ref):
    logits = lax.dot_general(
        x_ref[...], h_ref[...], (((1,), (1,)), ((), ())),
        preferred_element_type=jnp.float32)
    m = jnp.max(logits, axis=1, keepdims=True)
    e = jnp.exp(logits - m)
    o_ref[...] = e * (1.0 / jnp.sum(e, axis=1, keepdims=True))

  return pl.pallas_call(
      body,
      grid=(B // bb,),
      in_specs=[
          pl.BlockSpec((bb, Dp), lambda i: (i, 0)),
          pl.BlockSpec((C, Dp), lambda i: (0, 0)),
      ],
      out_specs=pl.BlockSpec((bb, C), lambda i: (i, 0)),
      out_shape=jax.ShapeDtypeStruct((B, C), jnp.float32),
  )(x, h)


def kernel(x, ancestor_idx, table):
  C, K = ancestor_idx.shape
  A, D = table.shape
  idx = ancestor_idx.astype(jnp.int32).reshape(-1)
  table_p = _pad_table_tc(table, _DP)
  x_p = jnp.pad(x, ((0, 0), (0, _DP - D)))
  local_h = _embed_bag_sc(C, K, A, _DP)(idx, table_p)
  return _matmul_softmax_tc(x_p, local_h, 128)
